# baseline (speedup 1.0000x reference)
import functools

import jax
import jax.numpy as jnp
from jax.experimental import pallas as pl


B, N, F = 256, 256, 53
NF = 36
BB = 8


def _fused_kernel(x_ref, wr_ref, br_ref, wc_ref, bc_ref, w1_ref, b1_ref,
                  w5_ref, b5_ref, w6_ref, b6_ref, w7_ref, b7_ref, out_ref):
    f32 = jnp.float32
    pooled = []
    for i in range(BB):
        f = x_ref[i, :, :NF]
        fb = f.astype(jnp.bfloat16)
        g = jax.lax.dot_general(fb, fb, (((1,), (1,)), ((), ())),
                                preferred_element_type=f32)
        a = jnp.maximum(g, 0.0)
        faug = jnp.concatenate(
            [fb, jnp.ones((N, 1), dtype=jnp.bfloat16)], axis=1)
        m = jax.lax.dot_general(a.astype(jnp.bfloat16), faug,
                                (((1,), (0,)), ((), ())),
                                preferred_element_type=f32)
        deg = m[:, NF:NF + 1] + 1e-6
        msg = m[:, :NF] / deg
        h = jax.lax.dot_general(msg.astype(jnp.bfloat16),
                                wr_ref[...].astype(jnp.bfloat16),
                                (((1,), (0,)), ((), ())),
                                preferred_element_type=f32)
        h = jnp.maximum(h + br_ref[...], 0.0)
        pooled.append(jnp.sum(h, axis=0, keepdims=True))
    p = jnp.concatenate(pooled, axis=0)

    def mm(x, w):
        return jax.lax.dot_general(x.astype(jnp.bfloat16),
                                   w.astype(jnp.bfloat16),
                                   (((1,), (0,)), ((), ())),
                                   preferred_element_type=f32)

    c = jnp.maximum(mm(p, wc_ref[...]) + bc_ref[...], 0.0)
    x1 = jnp.maximum(mm(c, w1_ref[...]) + b1_ref[...], 0.0)
    x5 = jnp.maximum(mm(x1, w5_ref[...]) + b5_ref[...], 0.0)
    mv = mm(x5, w6_ref[...]) + b6_ref[...]
    phys = x_ref[:, 0, NF + 2:F]
    w7 = w7_ref[...]
    col0 = mv * w7[0, 0] + jax.lax.dot_general(
        phys, w7[1:, :], (((1,), (0,)), ((), ())),
        preferred_element_type=f32) + b7_ref[...]
    out_ref[...] = jnp.concatenate([col0, phys], axis=1)


@functools.partial(jax.jit, static_argnames=())
def kernel(inputs, W_rule, b_rule, W_conv, b_conv, W1, b1, W5, b5, W6, b6,
           W7, b7):
    full = lambda shape: pl.BlockSpec(shape, lambda i: (0,) * len(shape))
    grid = B // BB
    out = pl.pallas_call(
        _fused_kernel,
        grid=(grid,),
        in_specs=[
            pl.BlockSpec((BB, N, F), lambda i: (i, 0, 0)),
            full((36, 20)),
            full((1, 20)),
            full((20, 1024)),
            full((1, 1024)),
            full((1024, 32)),
            full((1, 32)),
            full((32, 16)),
            full((1, 16)),
            full((16, 1)),
            full((1, 1)),
            full((16, 1)),
            full((1, 1)),
        ],
        out_specs=pl.BlockSpec((BB, 16), lambda i: (i, 0)),
        out_shape=jax.ShapeDtypeStruct((B, 16), jnp.float32),
    )(inputs, W_rule, b_rule.reshape(1, 20), W_conv, b_conv.reshape(1, 1024),
      W1, b1.reshape(1, 32), W5, b5.reshape(1, 16), W6, b6.reshape(1, 1),
      W7, b7.reshape(1, 1))
    return out
